# baseline (speedup 1.0000x reference)
import functools

import jax
import jax.numpy as jnp
from jax.experimental import pallas as pl
from jax.experimental.pallas import tpu as pltpu

EPS_ = 1e-6
_ROWS_PER_STEP = 16


def _loss_kernel(p_ref, g_ref, b_of_row_ref, w_ref, out_ref,
                 acc_t_ref, acc_v_ref, *, B, n_steps):
    step = pl.program_id(0)

    @pl.when(step == 0)
    def _():
        for i in range(B):
            acc_t_ref[i] = 0.0
            acc_v_ref[i] = 0.0

    for r in range(_ROWS_PER_STEP):
        p = p_ref[r]
        g = g_ref[r]
        pos = jnp.sum(g * p)
        s = jnp.sum(p)
        mx = jnp.max(g)
        row = step * _ROWS_PER_STEP + r
        b = b_of_row_ref[row]
        ratio = (s - pos) / (pos + EPS_)
        is_valid = mx != 0.0
        contrib = jnp.where(is_valid, ratio * w_ref[row], 0.0)
        acc_t_ref[b] = acc_t_ref[b] + contrib
        acc_v_ref[b] = jnp.maximum(acc_v_ref[b],
                                   is_valid.astype(jnp.float32))

    @pl.when(step == n_steps - 1)
    def _():
        total = jnp.float32(0.0)
        n_valid = jnp.float32(0.0)
        for i in range(B):
            total = total + acc_t_ref[i]
            n_valid = n_valid + acc_v_ref[i]
        n = jnp.maximum(n_valid, 1.0)
        out_ref[0] = jnp.where(total == 0.0, 0.0, jnp.log(total) / n)


@jax.jit
def kernel(Y_pred, Y_gt, label):
    B, C, H, W = Y_pred.shape
    label32 = label.astype(jnp.int32)
    n_rows = B * C
    n_steps = n_rows // _ROWS_PER_STEP
    Yp = Y_pred.reshape(n_rows, H * W // 128, 128)
    Yg = Y_gt.reshape(n_rows, H * W // 128, 128)

    rows = jnp.arange(n_rows, dtype=jnp.int32)
    b_of_row = rows // C
    c_of_row = rows % C
    w_of_row = jnp.where(label32[b_of_row] == c_of_row,
                         jnp.float32(1.0), jnp.float32(1.0 / C))

    out = pl.pallas_call(
        functools.partial(_loss_kernel, B=B, n_steps=n_steps),
        grid=(n_steps,),
        in_specs=[
            pl.BlockSpec((_ROWS_PER_STEP, H * W // 128, 128),
                         lambda g: (g, 0, 0)),
            pl.BlockSpec((_ROWS_PER_STEP, H * W // 128, 128),
                         lambda g: (g, 0, 0)),
            pl.BlockSpec(memory_space=pltpu.SMEM),
            pl.BlockSpec(memory_space=pltpu.SMEM),
        ],
        out_specs=pl.BlockSpec(memory_space=pltpu.SMEM),
        out_shape=jax.ShapeDtypeStruct((1,), jnp.float32),
        scratch_shapes=[
            pltpu.SMEM((B,), jnp.float32),
            pltpu.SMEM((B,), jnp.float32),
        ],
        compiler_params=pltpu.CompilerParams(
            dimension_semantics=("arbitrary",),
        ),
    )(Yp, Yg, b_of_row, w_of_row)
    return out[0]
